# Initial kernel scaffold; baseline (speedup 1.0000x reference)
import jax
import jax.numpy as jnp
from jax.experimental import pallas as pl


def kernel(x, edge_index):
    raise NotImplementedError("write your pallas kernel here")


# speedup vs baseline: 1.3877x; 1.3877x over previous
import functools

import jax
import jax.numpy as jnp
from jax import lax
from jax.experimental import pallas as pl
from jax.experimental.pallas import tpu as pltpu
from jax.experimental.pallas import tpu_sc as plsc

_K = 3
_M = 4
_L = 16


def _build_adj(src, dst, n, e):
    info = plsc.get_sparse_core_info()
    nw = info.num_cores * info.num_subcores
    rows = n // nw
    mesh = plsc.VectorSubcoreMesh(core_axis_name="c", subcore_axis_name="s")

    @functools.partial(
        pl.kernel,
        mesh=mesh,
        out_type=jax.ShapeDtypeStruct((n * n,), jnp.float32),
        scratch_types=[
            pltpu.VMEM((e,), jnp.int32),
            pltpu.VMEM((e,), jnp.int32),
            pltpu.VMEM((rows * n,), jnp.float32),
        ],
        compiler_params=pltpu.CompilerParams(needs_layout_passes=False),
    )
    def sc_scatter(src_hbm, dst_hbm, a_hbm, src_v, dst_v, slab):
        wid = lax.axis_index("s") * info.num_cores + lax.axis_index("c")
        base = wid * rows
        pltpu.sync_copy(src_hbm, src_v)
        pltpu.sync_copy(dst_hbm, dst_v)

        zeros = jnp.zeros((_L,), jnp.float32)

        def zero_body(i, carry):
            slab[pl.ds(i * _L, _L)] = zeros
            return carry

        lax.fori_loop(0, rows * n // _L, zero_body, 0)

        ones = jnp.ones((_L,), jnp.float32)

        def edge_body(i, carry):
            s = src_v[pl.ds(i * _L, _L)]
            d = dst_v[pl.ds(i * _L, _L)]
            m = (s >= base) & (s < base + rows)
            idx = jnp.where(m, (s - base) * n + d, 0)
            plsc.store_scatter(slab, [idx], ones, mask=m)
            return carry

        lax.fori_loop(0, e // _L, edge_body, 0)
        pltpu.sync_copy(slab, a_hbm.at[pl.ds(base * n, rows * n)])

    return sc_scatter(src, dst)


def _tc_body(a_ref, x_ref, out_ref):
    n = a_ref.shape[0]
    d = x_ref.shape[1]

    xa = jnp.abs(x_ref[...])
    x2 = xa * xa
    xcat = jnp.concatenate([xa, x2, x2 * xa, x2 * x2], axis=1)

    row = lax.broadcasted_iota(jnp.int32, (n, n), 0)
    col = lax.broadcasted_iota(jnp.int32, (n, n), 1)
    a = a_ref[...]
    s1 = jnp.where((row == col) | (a > 0.0), 1.0, 0.0).astype(jnp.bfloat16)

    c2 = lax.dot_general(s1, s1, (((1,), (0,)), ((), ())),
                         preferred_element_type=jnp.float32)
    s2 = (c2 > 0.0).astype(jnp.bfloat16)
    c3 = lax.dot_general(s1, s2, (((1,), (0,)), ((), ())),
                         preferred_element_type=jnp.float32)
    s3 = (c3 > 0.0).astype(jnp.bfloat16)

    for k, s in enumerate((s1, s2, s3)):
        out_ref[:, k, :] = lax.dot_general(
            s.astype(jnp.float32), xcat, (((0,), (0,)), ((), ())),
            preferred_element_type=jnp.float32)


def _tc_compute(a, x2d, n, d):
    return pl.pallas_call(
        _tc_body,
        out_shape=jax.ShapeDtypeStruct((n, _K, _M * d), jnp.float32),
    )(a, x2d)


def kernel(x, edge_index):
    b, n, d = x.shape
    e = edge_index.shape[1]
    a = _build_adj(edge_index[0], edge_index[1], n, e).reshape(n, n)
    outs = []
    for bi in range(b):
        o = _tc_compute(a, x[bi], n, d)
        outs.append(o.reshape(n, _K, _M, d))
    return jnp.stack(outs, axis=0)


# speedup vs baseline: 1.7759x; 1.2798x over previous
import functools

import jax
import jax.numpy as jnp
from jax import lax
from jax.experimental import pallas as pl
from jax.experimental.pallas import tpu as pltpu
from jax.experimental.pallas import tpu_sc as plsc

_K = 3
_M = 4
_L = 16


def _build_adj(src, dst, n, e):
    info = plsc.get_sparse_core_info()
    nw = info.num_cores * info.num_subcores
    rows = n // nw
    mesh = plsc.VectorSubcoreMesh(core_axis_name="c", subcore_axis_name="s")

    @functools.partial(
        pl.kernel,
        mesh=mesh,
        out_type=jax.ShapeDtypeStruct((n, n), jnp.float32),
        scratch_types=[
            pltpu.VMEM((e,), jnp.int32),
            pltpu.VMEM((e,), jnp.int32),
            pltpu.VMEM((rows, n), jnp.float32),
        ],
        compiler_params=pltpu.CompilerParams(needs_layout_passes=False),
    )
    def sc_scatter(src_hbm, dst_hbm, a_hbm, src_v, dst_v, slab):
        wid = lax.axis_index("s") * info.num_cores + lax.axis_index("c")
        base = wid * rows
        pltpu.sync_copy(src_hbm, src_v)
        pltpu.sync_copy(dst_hbm, dst_v)

        zeros = jnp.zeros((_L,), jnp.float32)
        for r in range(rows):
            @plsc.parallel_loop(0, n // _L, 1, unroll=8)
            def _(j, r=r):
                slab[r, pl.ds(j * _L, _L)] = zeros

        ones = jnp.ones((_L,), jnp.float32)

        @plsc.parallel_loop(0, e // _L, 1, unroll=4)
        def _(i):
            s = src_v[pl.ds(i * _L, _L)]
            d = dst_v[pl.ds(i * _L, _L)]
            m = (s >= base) & (s < base + rows)
            r = jnp.where(m, s - base, 0)
            plsc.store_scatter(slab, [r, d], ones, mask=m)

        pltpu.sync_copy(slab, a_hbm.at[pl.ds(base, rows)])

    return sc_scatter(src, dst)


def _tc_body(a_ref, x_ref, out_ref):
    n = a_ref.shape[0]
    d = x_ref.shape[1]

    xa = jnp.abs(x_ref[...])
    x2 = xa * xa
    xcat = jnp.concatenate([xa, x2, x2 * xa, x2 * x2], axis=1)
    xcat = xcat.astype(jnp.bfloat16)

    row = lax.broadcasted_iota(jnp.int32, (n, n), 0)
    col = lax.broadcasted_iota(jnp.int32, (n, n), 1)
    a = a_ref[...]
    s1 = jnp.where((row == col) | (a > 0.0), 1.0, 0.0).astype(jnp.bfloat16)

    c2 = lax.dot_general(s1, s1, (((1,), (0,)), ((), ())),
                         preferred_element_type=jnp.float32)
    s2 = (c2 > 0.0).astype(jnp.bfloat16)
    c3 = lax.dot_general(s1, s2, (((1,), (0,)), ((), ())),
                         preferred_element_type=jnp.float32)
    s3 = (c3 > 0.0).astype(jnp.bfloat16)

    for k, s in enumerate((s1, s2, s3)):
        out_ref[:, k, :] = lax.dot_general(
            s, xcat, (((0,), (0,)), ((), ())),
            preferred_element_type=jnp.float32)


def _tc_compute(a, x2d, n, d):
    return pl.pallas_call(
        _tc_body,
        out_shape=jax.ShapeDtypeStruct((n, _K, _M * d), jnp.float32),
    )(a, x2d)


def kernel(x, edge_index):
    b, n, d = x.shape
    e = edge_index.shape[1]
    a = _build_adj(edge_index[0], edge_index[1], n, e)
    outs = []
    for bi in range(b):
        o = _tc_compute(a, x[bi], n, d)
        outs.append(o.reshape(n, _K, _M, d))
    return jnp.stack(outs, axis=0)
